# baseline (speedup 1.0000x reference)
import functools

import jax
import jax.numpy as jnp
from jax import lax
from jax.experimental import pallas as pl
from jax.experimental.pallas import tpu as pltpu
from jax.experimental.pallas import tpu_sc as plsc

D = 128
DM = 2 * D
B = 16384

NC = 2
NS = 16
NW = NC * NS
NSLICE = 2
RB = B // NSLICE
BPW = RB // NW
CH = 128
NCH = BPW // CH
CH2 = 64
NCH2 = BPW // CH2

_RND = 0x7FFF


def _rne_word(a, b):
  au = lax.bitcast_convert_type(a, jnp.int32)
  bu = lax.bitcast_convert_type(b, jnp.int32)
  al = lax.shift_right_logical(
      au + _RND + (lax.shift_right_logical(au, 16) & 1), 16)
  bh = (bu + _RND + (lax.shift_right_logical(bu, 16) & 1)) & (-65536)
  return al | bh


def _sc_gather(s, inputs, eu_gmf, ei_gmf, eu_mlp, ei_mlp):
  mesh = plsc.VectorSubcoreMesh(core_axis_name="c", subcore_axis_name="s")
  sbase = s * RB

  @functools.partial(
      pl.kernel,
      out_type=(
          jax.ShapeDtypeStruct((RB // 2, DM), jnp.int32),
          jax.ShapeDtypeStruct((RB // 2, DM), jnp.int32),
          jax.ShapeDtypeStruct((RB, D), jnp.float32),
      ),
      mesh=mesh,
      scratch_types=[
          pltpu.VMEM((BPW,), jnp.int32),
          pltpu.VMEM((BPW,), jnp.int32),
          pltpu.VMEM((CH, DM), jnp.float32),
          pltpu.VMEM((CH, DM), jnp.float32),
          pltpu.VMEM((CH // 2, DM), jnp.int32),
          pltpu.VMEM((CH // 2, DM), jnp.int32),
          pltpu.VMEM((CH2, D), jnp.float32),
          pltpu.VMEM((CH2, D), jnp.float32),
          pltpu.VMEM((CH2, D), jnp.float32),
          pltpu.SemaphoreType.DMA,
          pltpu.SemaphoreType.DMA,
          pltpu.SemaphoreType.DMA,
          pltpu.SemaphoreType.DMA,
      ],
  )
  def body(idx_hbm, eu_gmf_h, ei_gmf_h, eu_mlp_h, ei_mlp_h,
           um_out, im_out, gmf_out,
           idx_u, idx_s, bw0, bw1, pw0, pw1, bu, bi, pg,
           g0, g1, w0, w1):
    wid = lax.axis_index("s") * NC + lax.axis_index("c")
    base = wid * BPW
    pltpu.sync_copy(idx_hbm.at[0, pl.ds(sbase + base, BPW)], idx_u)
    pltpu.sync_copy(idx_hbm.at[1, pl.ds(sbase + base, BPW)], idx_s)

    def pack_chunk(src, dst):
      def pair(q, carry):
        for g in range(DM // 16):
          a = src[2 * q, pl.ds(16 * g, 16)]
          b = src[2 * q + 1, pl.ds(16 * g, 16)]
          dst[q, pl.ds(16 * g, 16)] = _rne_word(a, b)
        return carry
      lax.fori_loop(0, CH // 2, pair, 0)

    tasks = [(tbl, idx, out, ci)
             for tbl, idx, out in ((eu_mlp_h, idx_u, um_out),
                                   (ei_mlp_h, idx_s, im_out))
             for ci in range(NCH)]
    bufs, pbufs = (bw0, bw1), (pw0, pw1)
    gsems, wsems = (g0, g1), (w0, w1)
    gdescs = [None, None]
    wdescs = [None, None]

    def start(t):
      tbl, idx, out, ci = tasks[t]
      slot = t % 2
      gdescs[slot] = pltpu.async_copy(
          tbl.at[idx.at[pl.ds(ci * CH, CH)]], bufs[slot], gsems[slot])

    start(0)
    for t in range(len(tasks)):
      slot = t % 2
      if t + 1 < len(tasks):
        start(t + 1)
      gdescs[slot].wait()
      if wdescs[slot] is not None:
        wdescs[slot].wait()
      pack_chunk(bufs[slot], pbufs[slot])
      _, _, out, ci = tasks[t]
      wdescs[slot] = pltpu.async_copy(
          pbufs[slot],
          out.at[pl.ds(pl.multiple_of((base + ci * CH) // 2, CH // 2),
                       CH // 2)],
          wsems[slot])
    for d in wdescs:
      if d is not None:
        d.wait()

    def mul_chunk():
      def row(r, carry):
        for g in range(D // 16):
          pg[r, pl.ds(16 * g, 16)] = (
              bu[r, pl.ds(16 * g, 16)] * bi[r, pl.ds(16 * g, 16)])
        return carry
      lax.fori_loop(0, CH2, row, 0)

    def gstart(ci):
      return (pltpu.async_copy(
                  eu_gmf_h.at[idx_u.at[pl.ds(ci * CH2, CH2)]], bu, g0),
              pltpu.async_copy(
                  ei_gmf_h.at[idx_s.at[pl.ds(ci * CH2, CH2)]], bi, g1))

    pend = gstart(0)
    wprev = None
    for t in range(NCH2):
      pend[0].wait()
      pend[1].wait()
      if wprev is not None:
        wprev.wait()
      mul_chunk()
      if t + 1 < NCH2:
        pend = gstart(t + 1)
      wprev = pltpu.async_copy(
          pg, gmf_out.at[pl.ds(base + t * CH2, CH2)], w0)
    wprev.wait()

  return body(inputs, eu_gmf, ei_gmf, eu_mlp, ei_mlp)


def _dense_body(um_ref, im_ref, gmf_ref,
                w1u_ref, w1i_ref, b1_ref, w2_ref, b2_ref, wp_ref, bp_ref,
                *rest):
  emb_ref, y_ref = rest[-2], rest[-1]
  bf = jnp.bfloat16
  um = pltpu.bitcast(um_ref[...], bf)
  im = pltpu.bitcast(im_ref[...], bf)
  h = jnp.dot(um, w1u_ref[...].astype(bf), preferred_element_type=jnp.float32)
  h += jnp.dot(im, w1i_ref[...].astype(bf), preferred_element_type=jnp.float32)
  h = jnp.maximum(h + b1_ref[...], 0.0)
  h2 = jnp.dot(h.astype(bf), w2_ref[...].astype(bf),
               preferred_element_type=jnp.float32)
  h2 = jnp.maximum(h2 + b2_ref[...], 0.0)
  emb = jnp.concatenate([gmf_ref[...], h2], axis=-1)
  emb_ref[...] = emb
  y_ref[...] = jnp.sum(emb * wp_ref[...], axis=-1) + bp_ref[0]


def _tc_dense_slice(s, um, im, gmf, w1u, w1i, b1r, w2, b2r, wpr, bpr,
                    emb_in, y_in):
  bs = 2048
  nblk = RB // bs
  in_specs = [
      pl.BlockSpec((bs // 2, DM), lambda i: (i, 0)),
      pl.BlockSpec((bs // 2, DM), lambda i: (i, 0)),
      pl.BlockSpec((bs, D), lambda i: (i, 0)),
      pl.BlockSpec((DM, DM), lambda i: (0, 0)),
      pl.BlockSpec((DM, DM), lambda i: (0, 0)),
      pl.BlockSpec((1, DM), lambda i: (0, 0)),
      pl.BlockSpec((DM, D), lambda i: (0, 0)),
      pl.BlockSpec((1, D), lambda i: (0, 0)),
      pl.BlockSpec((1, DM), lambda i: (0, 0)),
      pl.BlockSpec(memory_space=pltpu.SMEM),
  ]
  args = [um, im, gmf, w1u, w1i, b1r, w2, b2r, wpr, bpr]
  aliases = {}
  if s > 0:
    in_specs += [pl.BlockSpec(memory_space=pl.ANY),
                 pl.BlockSpec(memory_space=pl.ANY)]
    args += [emb_in, y_in]
    aliases = {10: 0, 11: 1}
  off = s * nblk
  return pl.pallas_call(
      _dense_body,
      grid=(nblk,),
      in_specs=in_specs,
      out_specs=[
          pl.BlockSpec((bs, DM), lambda i: (i + off, 0)),
          pl.BlockSpec((bs,), lambda i: (i + off,)),
      ],
      out_shape=[
          jax.ShapeDtypeStruct((B, DM), jnp.float32),
          jax.ShapeDtypeStruct((B,), jnp.float32),
      ],
      input_output_aliases=aliases,
  )(*args)


def kernel(inputs, eu_gmf, ei_gmf, eu_mlp, ei_mlp, W1, b1, W2, b2, Wp, bp):
  w1t = W1.T
  w1u = w1t[:DM]
  w1i = w1t[DM:]
  w2 = W2.T
  b1r = b1.reshape(1, -1)
  b2r = b2.reshape(1, -1)
  wpr = Wp.reshape(1, -1)
  bpr = bp.reshape(1)
  emb, y = None, None
  for s in range(NSLICE):
    um, im, gmf = _sc_gather(s, inputs, eu_gmf, ei_gmf, eu_mlp, ei_mlp)
    emb, y = _tc_dense_slice(s, um, im, gmf,
                             w1u, w1i, b1r, w2, b2r, wpr, bpr, emb, y)
  return emb, y


# speedup vs baseline: 1.4932x; 1.4932x over previous
import functools

import jax
import jax.numpy as jnp
from jax import lax
from jax.experimental import pallas as pl
from jax.experimental.pallas import tpu as pltpu
from jax.experimental.pallas import tpu_sc as plsc

D = 128
DM = 2 * D
B = 16384

NC = 2
NS = 16
NW = NC * NS
CH = 128
NSLICE = 4
RB = B // NSLICE
BPW = RB // NW
NCH = BPW // CH


def _sc_gather(s, inputs, eu_gmf, ei_gmf, eu_mlp, ei_mlp):
  mesh = plsc.VectorSubcoreMesh(core_axis_name="c", subcore_axis_name="s")
  sbase = s * RB

  @functools.partial(
      pl.kernel,
      out_type=(
          jax.ShapeDtypeStruct((RB, DM), jnp.float32),
          jax.ShapeDtypeStruct((RB, DM), jnp.float32),
          jax.ShapeDtypeStruct((RB, D), jnp.float32),
          jax.ShapeDtypeStruct((RB, D), jnp.float32),
      ),
      mesh=mesh,
      scratch_types=[
          pltpu.VMEM((BPW,), jnp.int32),
          pltpu.VMEM((BPW,), jnp.int32),
          pltpu.VMEM((CH, DM), jnp.float32),
          pltpu.VMEM((CH, DM), jnp.float32),
          pltpu.VMEM((CH, D), jnp.float32),
          pltpu.VMEM((CH, D), jnp.float32),
          pltpu.SemaphoreType.DMA,
          pltpu.SemaphoreType.DMA,
      ],
  )
  def body(idx_hbm, eu_gmf_h, ei_gmf_h, eu_mlp_h, ei_mlp_h,
           um_out, im_out, ug_out, ig_out,
           idx_u, idx_s, bw0, bw1, bn0, bn1, sem0, sem1):
    wid = lax.axis_index("s") * NC + lax.axis_index("c")
    base = wid * BPW
    pltpu.sync_copy(idx_hbm.at[0, pl.ds(sbase + base, BPW)], idx_u)
    pltpu.sync_copy(idx_hbm.at[1, pl.ds(sbase + base, BPW)], idx_s)
    sems = (sem0, sem1)

    def run_ring(tables, bufs):
      tasks = [(tbl, idx, out, ci)
               for tbl, idx, out in tables for ci in range(NCH)]
      descs = [None, None]

      def start(t):
        tbl, idx, out, ci = tasks[t]
        slot = t % 2
        descs[slot] = pltpu.async_copy(
            tbl.at[idx.at[pl.ds(ci * CH, CH)]], bufs[slot], sems[slot])

      start(0)
      for t in range(len(tasks)):
        slot = t % 2
        if t + 1 < len(tasks):
          start(t + 1)
        descs[slot].wait()
        _, _, out, ci = tasks[t]
        pltpu.sync_copy(bufs[slot], out.at[pl.ds(base + ci * CH, CH)])

    run_ring(((eu_mlp_h, idx_u, um_out), (ei_mlp_h, idx_s, im_out)),
             (bw0, bw1))
    run_ring(((eu_gmf_h, idx_u, ug_out), (ei_gmf_h, idx_s, ig_out)),
             (bn0, bn1))

  return body(inputs, eu_gmf, ei_gmf, eu_mlp, ei_mlp)


def _dense_body(um_ref, im_ref, ug_ref, ig_ref,
                w1u_ref, w1i_ref, b1_ref, w2_ref, b2_ref, wp_ref, bp_ref,
                *rest):
  emb_ref, y_ref = rest[-2], rest[-1]
  bf = jnp.bfloat16
  h = jnp.dot(um_ref[...].astype(bf), w1u_ref[...].astype(bf),
              preferred_element_type=jnp.float32)
  h += jnp.dot(im_ref[...].astype(bf), w1i_ref[...].astype(bf),
               preferred_element_type=jnp.float32)
  h = jnp.maximum(h + b1_ref[...], 0.0)
  h2 = jnp.dot(h.astype(bf), w2_ref[...].astype(bf),
               preferred_element_type=jnp.float32)
  h2 = jnp.maximum(h2 + b2_ref[...], 0.0)
  gmf = ug_ref[...] * ig_ref[...]
  emb = jnp.concatenate([gmf, h2], axis=-1)
  emb_ref[...] = emb
  y_ref[...] = jnp.sum(emb * wp_ref[...], axis=-1) + bp_ref[0]


def _tc_dense_slice(s, um, im, ug, ig, w1u, w1i, b1r, w2, b2r, wpr, bpr,
                    emb_in, y_in):
  bs = 2048
  nblk = RB // bs
  in_specs = [
      pl.BlockSpec((bs, DM), lambda i: (i, 0)),
      pl.BlockSpec((bs, DM), lambda i: (i, 0)),
      pl.BlockSpec((bs, D), lambda i: (i, 0)),
      pl.BlockSpec((bs, D), lambda i: (i, 0)),
      pl.BlockSpec((DM, DM), lambda i: (0, 0)),
      pl.BlockSpec((DM, DM), lambda i: (0, 0)),
      pl.BlockSpec((1, DM), lambda i: (0, 0)),
      pl.BlockSpec((DM, D), lambda i: (0, 0)),
      pl.BlockSpec((1, D), lambda i: (0, 0)),
      pl.BlockSpec((1, DM), lambda i: (0, 0)),
      pl.BlockSpec(memory_space=pltpu.SMEM),
  ]
  args = [um, im, ug, ig, w1u, w1i, b1r, w2, b2r, wpr, bpr]
  aliases = {}
  if s > 0:
    in_specs += [pl.BlockSpec(memory_space=pl.ANY),
                 pl.BlockSpec(memory_space=pl.ANY)]
    args += [emb_in, y_in]
    aliases = {11: 0, 12: 1}
  off = s * nblk
  return pl.pallas_call(
      _dense_body,
      grid=(nblk,),
      in_specs=in_specs,
      out_specs=[
          pl.BlockSpec((bs, DM), lambda i: (i + off, 0)),
          pl.BlockSpec((bs,), lambda i: (i + off,)),
      ],
      out_shape=[
          jax.ShapeDtypeStruct((B, DM), jnp.float32),
          jax.ShapeDtypeStruct((B,), jnp.float32),
      ],
      input_output_aliases=aliases,
  )(*args)


def kernel(inputs, eu_gmf, ei_gmf, eu_mlp, ei_mlp, W1, b1, W2, b2, Wp, bp):
  w1t = W1.T
  w1u = w1t[:DM]
  w1i = w1t[DM:]
  w2 = W2.T
  b1r = b1.reshape(1, -1)
  b2r = b2.reshape(1, -1)
  wpr = Wp.reshape(1, -1)
  bpr = bp.reshape(1)
  emb, y = None, None
  for s in range(NSLICE):
    um, im, ug, ig = _sc_gather(s, inputs, eu_gmf, ei_gmf, eu_mlp, ei_mlp)
    emb, y = _tc_dense_slice(s, um, im, ug, ig,
                             w1u, w1i, b1r, w2, b2r, wpr, bpr, emb, y)
  return emb, y


# speedup vs baseline: 1.5102x; 1.0114x over previous
import functools

import jax
import jax.numpy as jnp
from jax import lax
from jax.experimental import pallas as pl
from jax.experimental.pallas import tpu as pltpu
from jax.experimental.pallas import tpu_sc as plsc

D = 128
DM = 2 * D
B = 16384

NC = 2
NS = 16
NW = NC * NS
CH = 64
NBUF = 3
NSLICE = 2
RB = B // NSLICE
BPW = RB // NW
NCH = BPW // CH


def _sc_gather(s, inputs, eu_gmf, ei_gmf, eu_mlp, ei_mlp):
  mesh = plsc.VectorSubcoreMesh(core_axis_name="c", subcore_axis_name="s")
  sbase = s * RB

  @functools.partial(
      pl.kernel,
      out_type=(
          jax.ShapeDtypeStruct((RB, DM), jnp.float32),
          jax.ShapeDtypeStruct((RB, DM), jnp.float32),
          jax.ShapeDtypeStruct((RB, D), jnp.float32),
          jax.ShapeDtypeStruct((RB, D), jnp.float32),
      ),
      mesh=mesh,
      scratch_types=[
          pltpu.VMEM((BPW,), jnp.int32),
          pltpu.VMEM((BPW,), jnp.int32),
          pltpu.VMEM((CH, DM), jnp.float32),
          pltpu.VMEM((CH, DM), jnp.float32),
          pltpu.VMEM((CH, DM), jnp.float32),
          pltpu.VMEM((CH, D), jnp.float32),
          pltpu.VMEM((CH, D), jnp.float32),
          pltpu.VMEM((CH, D), jnp.float32),
          pltpu.SemaphoreType.DMA,
          pltpu.SemaphoreType.DMA,
          pltpu.SemaphoreType.DMA,
      ],
  )
  def body(idx_hbm, eu_gmf_h, ei_gmf_h, eu_mlp_h, ei_mlp_h,
           um_out, im_out, ug_out, ig_out,
           idx_u, idx_s, bw0, bw1, bw2, bn0, bn1, bn2, sem0, sem1, sem2):
    wid = lax.axis_index("s") * NC + lax.axis_index("c")
    base = wid * BPW
    pltpu.sync_copy(idx_hbm.at[0, pl.ds(sbase + base, BPW)], idx_u)
    pltpu.sync_copy(idx_hbm.at[1, pl.ds(sbase + base, BPW)], idx_s)
    sems = (sem0, sem1, sem2)

    def run_ring(tables, bufs):
      tasks = [(tbl, idx, out, ci)
               for tbl, idx, out in tables for ci in range(NCH)]
      descs = [None] * NBUF

      def start(t):
        tbl, idx, out, ci = tasks[t]
        slot = t % NBUF
        descs[slot] = pltpu.async_copy(
            tbl.at[idx.at[pl.ds(ci * CH, CH)]], bufs[slot], sems[slot])

      for t in range(min(NBUF - 1, len(tasks))):
        start(t)
      for t in range(len(tasks)):
        slot = t % NBUF
        descs[slot].wait()
        if t + NBUF - 1 < len(tasks):
          start(t + NBUF - 1)
        _, _, out, ci = tasks[t]
        pltpu.sync_copy(bufs[slot], out.at[pl.ds(base + ci * CH, CH)])

    run_ring(((eu_mlp_h, idx_u, um_out), (ei_mlp_h, idx_s, im_out)),
             (bw0, bw1, bw2))
    run_ring(((eu_gmf_h, idx_u, ug_out), (ei_gmf_h, idx_s, ig_out)),
             (bn0, bn1, bn2))

  return body(inputs, eu_gmf, ei_gmf, eu_mlp, ei_mlp)


def _dense_body(um_ref, im_ref, ug_ref, ig_ref,
                w1u_ref, w1i_ref, b1_ref, w2_ref, b2_ref, wp_ref, bp_ref,
                *rest):
  emb_ref, y_ref = rest[-2], rest[-1]
  bf = jnp.bfloat16
  h = jnp.dot(um_ref[...].astype(bf), w1u_ref[...].astype(bf),
              preferred_element_type=jnp.float32)
  h += jnp.dot(im_ref[...].astype(bf), w1i_ref[...].astype(bf),
               preferred_element_type=jnp.float32)
  h = jnp.maximum(h + b1_ref[...], 0.0)
  h2 = jnp.dot(h.astype(bf), w2_ref[...].astype(bf),
               preferred_element_type=jnp.float32)
  h2 = jnp.maximum(h2 + b2_ref[...], 0.0)
  gmf = ug_ref[...] * ig_ref[...]
  emb = jnp.concatenate([gmf, h2], axis=-1)
  emb_ref[...] = emb
  y_ref[...] = jnp.sum(emb * wp_ref[...], axis=-1) + bp_ref[0]


def _tc_dense_slice(s, um, im, ug, ig, w1u, w1i, b1r, w2, b2r, wpr, bpr,
                    emb_in, y_in):
  bs = 2048
  nblk = RB // bs
  in_specs = [
      pl.BlockSpec((bs, DM), lambda i: (i, 0)),
      pl.BlockSpec((bs, DM), lambda i: (i, 0)),
      pl.BlockSpec((bs, D), lambda i: (i, 0)),
      pl.BlockSpec((bs, D), lambda i: (i, 0)),
      pl.BlockSpec((DM, DM), lambda i: (0, 0)),
      pl.BlockSpec((DM, DM), lambda i: (0, 0)),
      pl.BlockSpec((1, DM), lambda i: (0, 0)),
      pl.BlockSpec((DM, D), lambda i: (0, 0)),
      pl.BlockSpec((1, D), lambda i: (0, 0)),
      pl.BlockSpec((1, DM), lambda i: (0, 0)),
      pl.BlockSpec(memory_space=pltpu.SMEM),
  ]
  args = [um, im, ug, ig, w1u, w1i, b1r, w2, b2r, wpr, bpr]
  aliases = {}
  if s > 0:
    in_specs += [pl.BlockSpec(memory_space=pl.ANY),
                 pl.BlockSpec(memory_space=pl.ANY)]
    args += [emb_in, y_in]
    aliases = {11: 0, 12: 1}
  off = s * nblk
  return pl.pallas_call(
      _dense_body,
      grid=(nblk,),
      in_specs=in_specs,
      out_specs=[
          pl.BlockSpec((bs, DM), lambda i: (i + off, 0)),
          pl.BlockSpec((bs,), lambda i: (i + off,)),
      ],
      out_shape=[
          jax.ShapeDtypeStruct((B, DM), jnp.float32),
          jax.ShapeDtypeStruct((B,), jnp.float32),
      ],
      input_output_aliases=aliases,
  )(*args)


def kernel(inputs, eu_gmf, ei_gmf, eu_mlp, ei_mlp, W1, b1, W2, b2, Wp, bp):
  w1t = W1.T
  w1u = w1t[:DM]
  w1i = w1t[DM:]
  w2 = W2.T
  b1r = b1.reshape(1, -1)
  b2r = b2.reshape(1, -1)
  wpr = Wp.reshape(1, -1)
  bpr = bp.reshape(1)
  emb, y = None, None
  for s in range(NSLICE):
    um, im, ug, ig = _sc_gather(s, inputs, eu_gmf, ei_gmf, eu_mlp, ei_mlp)
    emb, y = _tc_dense_slice(s, um, im, ug, ig,
                             w1u, w1i, b1r, w2, b2r, wpr, bpr, emb, y)
  return emb, y
